# Initial kernel scaffold; baseline (speedup 1.0000x reference)
import jax
import jax.numpy as jnp
from jax.experimental import pallas as pl


def kernel(x, fc0_w, fc0_b, fc1_w, fc1_b, ew1, eb1, ew2, eb2):
    raise NotImplementedError("write your pallas kernel here")


# speedup vs baseline: 1.1308x; 1.1308x over previous
import functools

import jax
import jax.numpy as jnp
from jax.experimental import pallas as pl
from jax.experimental.pallas import tpu as pltpu

E = 6
TOP_K = 3
B, C, H, W = 2, 192, 224, 224
HW = H * W
NBC = B * C


def _pool_body(x_ref, out_ref):
    xb = x_ref[...]
    mx = jnp.max(xb, axis=1, keepdims=True)
    sm = jnp.sum(xb, axis=1, keepdims=True)
    out_ref[...] = mx + sm * (1.0 / HW)


def _pool(x2):
    rows = 48
    grid = (NBC // rows,)
    return pl.pallas_call(
        _pool_body,
        grid=grid,
        in_specs=[pl.BlockSpec((rows, HW), lambda i: (i, 0))],
        out_specs=pl.BlockSpec((rows, 1), lambda i: (i, 0)),
        out_shape=jax.ShapeDtypeStruct((NBC, 1), jnp.float32),
    )(x2)


def _gate_body(pooled_ref, fc0_w_ref, fc0_b_ref, fc1_w_ref, fc1_b_ref,
               cof_ref, idx_ref):
    pooled = pooled_ref[...]
    dn = (((1,), (1,)), ((), ()))
    g = jax.lax.dot_general(pooled, fc1_w_ref[...], dn,
                            preferred_element_type=jnp.float32)
    g = g + fc1_b_ref[...]
    g = jnp.where(g > 0, g, 0.2 * g)

    z = jax.lax.dot_general(pooled, fc0_w_ref[...], dn,
                            preferred_element_type=jnp.float32)
    z = z + fc0_b_ref[...]
    noise = jnp.maximum(z, 0.0) + jnp.log1p(jnp.exp(-jnp.abs(z)))
    nmean = jnp.sum(noise, axis=1, keepdims=True) * (1.0 / E)
    dev = noise - nmean
    std = jnp.sqrt(jnp.sum(dev * dev, axis=1, keepdims=True) * (1.0 / (E - 1)))
    s = g + dev / std

    lane = jax.lax.broadcasted_iota(jnp.int32, (B, E), 1)
    rank = jnp.zeros((B, E), jnp.int32)
    for j in range(E):
        sj = s[:, j:j + 1]
        beats = (sj > s) | ((sj == s) & (j < lane))
        rank = rank + beats.astype(jnp.int32)
    mask = rank < TOP_K

    neg = jnp.float32(-1e30)
    gm = jnp.where(mask, g, neg)
    m = jnp.max(gm, axis=1, keepdims=True)
    ex = jnp.where(mask, jnp.exp(g - m), 0.0)
    cof_ref[...] = ex / jnp.sum(ex, axis=1, keepdims=True)

    cols = []
    for k in range(TOP_K):
        sel = (rank == k).astype(jnp.int32)
        cols.append(jnp.sum(sel * lane, axis=1, keepdims=True))
    idx_ref[...] = jnp.concatenate(cols, axis=1)


def _gate(pooled, fc0_w, fc0_b, fc1_w, fc1_b):
    return pl.pallas_call(
        _gate_body,
        in_specs=[
            pl.BlockSpec((B, C), lambda: (0, 0)),
            pl.BlockSpec((E, C), lambda: (0, 0)),
            pl.BlockSpec((1, E), lambda: (0, 0)),
            pl.BlockSpec((E, C), lambda: (0, 0)),
            pl.BlockSpec((1, E), lambda: (0, 0)),
        ],
        out_specs=[
            pl.BlockSpec((B, E), lambda: (0, 0)),
            pl.BlockSpec((B, TOP_K), lambda: (0, 0)),
        ],
        out_shape=[
            jax.ShapeDtypeStruct((B, E), jnp.float32),
            jax.ShapeDtypeStruct((B, TOP_K), jnp.int32),
        ],
    )(pooled, fc0_w, fc0_b.reshape(1, E), fc1_w, fc1_b.reshape(1, E))


def _conv_body(idx_ref, x_ref, ew1_ref, eb1_ref, ew2_ref, eb2_ref, cof_ref,
               out_ref, xp, hp):
    i = pl.program_id(0)
    k = pl.program_id(1)
    b = i // C

    @pl.when(jnp.logical_and(i == 0, k == 0))
    def _init():
        xp[...] = jnp.zeros_like(xp)
        hp[...] = jnp.zeros_like(hp)

    @pl.when(k == 0)
    def _fill():
        xp[1:H + 1, 1:W + 1] = x_ref[0]

    w1 = ew1_ref[0, 0, 0, :]
    b1 = eb1_ref[0, 0, 0, 0]
    h = w1[0] * xp[0:H, 0:W]
    for t in range(1, 9):
        dy, dx = t // 3, t % 3
        h = h + w1[t] * xp[dy:dy + H, dx:dx + W]
    hp[1:H + 1, 1:W + 1] = jnp.maximum(h + b1, 0.0)

    w2 = ew2_ref[0, 0, 0, :]
    b2 = eb2_ref[0, 0, 0, 0]
    y = w2[0] * hp[0:H, 0:W]
    for t in range(1, 9):
        dy, dx = t // 3, t % 3
        y = y + w2[t] * hp[dy:dy + H, dx:dx + W]
    y = y + b2

    e = idx_ref[b, k]
    coef = cof_ref[b, e]
    contrib = coef * y

    @pl.when(k == 0)
    def _set():
        out_ref[0] = contrib

    @pl.when(k > 0)
    def _acc():
        out_ref[0] = out_ref[0] + contrib


def _conv(idx, x3, ew1v, eb1v, ew2v, eb2v, cof):
    def widx(i, k, idx_ref):
        return (idx_ref[i // C, k], i % C, 0, 0)

    grid_spec = pltpu.PrefetchScalarGridSpec(
        num_scalar_prefetch=1,
        grid=(NBC, TOP_K),
        in_specs=[
            pl.BlockSpec((1, H, W), lambda i, k, idx_ref: (i, 0, 0)),
            pl.BlockSpec((1, 1, 1, 9), widx),
            pl.BlockSpec((1, 1, 1, 1), widx),
            pl.BlockSpec((1, 1, 1, 9), widx),
            pl.BlockSpec((1, 1, 1, 1), widx),
            pl.BlockSpec(memory_space=pltpu.SMEM),
        ],
        out_specs=pl.BlockSpec((1, H, W), lambda i, k, idx_ref: (i, 0, 0)),
        scratch_shapes=[
            pltpu.VMEM((H + 2, W + 2), jnp.float32),
            pltpu.VMEM((H + 2, W + 2), jnp.float32),
        ],
    )
    return pl.pallas_call(
        _conv_body,
        grid_spec=grid_spec,
        out_shape=jax.ShapeDtypeStruct((NBC, H, W), jnp.float32),
    )(idx, x3, ew1v, eb1v, ew2v, eb2v, cof)


@jax.jit
def kernel(x, fc0_w, fc0_b, fc1_w, fc1_b, ew1, eb1, ew2, eb2):
    x2 = x.reshape(NBC, HW)
    pooled = _pool(x2).reshape(B, C)
    cof, idx = _gate(pooled, fc0_w, fc0_b, fc1_w, fc1_b)

    x3 = x.reshape(NBC, H, W)
    ew1v = ew1.reshape(E, C, 1, 9)
    eb1v = eb1.reshape(E, C, 1, 1)
    ew2v = ew2.reshape(E, C, 1, 9)
    eb2v = eb2.reshape(E, C, 1, 1)
    out = _conv(idx, x3, ew1v, eb1v, ew2v, eb2v, cof)
    return out.reshape(B, C, H, W)


# speedup vs baseline: 3.7358x; 3.3037x over previous
import functools

import jax
import jax.numpy as jnp
from jax.experimental import pallas as pl
from jax.experimental.pallas import tpu as pltpu

E = 6
TOP_K = 3
B, C, H, W = 2, 192, 224, 224
HW = H * W
NBC = B * C


def _pool_body(x_ref, out_ref):
    xb = x_ref[...]
    mx = jnp.max(xb, axis=1, keepdims=True)
    sm = jnp.sum(xb, axis=1, keepdims=True)
    out_ref[...] = mx + sm * (1.0 / HW)


def _pool(x2):
    rows = 48
    grid = (NBC // rows,)
    return pl.pallas_call(
        _pool_body,
        grid=grid,
        in_specs=[pl.BlockSpec((rows, HW), lambda i: (i, 0))],
        out_specs=pl.BlockSpec((rows, 1), lambda i: (i, 0)),
        out_shape=jax.ShapeDtypeStruct((NBC, 1), jnp.float32),
    )(x2)


def _gate_body(pooled_ref, fc0_w_ref, fc0_b_ref, fc1_w_ref, fc1_b_ref,
               cof_ref, idx_ref):
    pooled = pooled_ref[...]
    dn = (((1,), (1,)), ((), ()))
    g = jax.lax.dot_general(pooled, fc1_w_ref[...], dn,
                            preferred_element_type=jnp.float32)
    g = g + fc1_b_ref[...]
    g = jnp.where(g > 0, g, 0.2 * g)

    z = jax.lax.dot_general(pooled, fc0_w_ref[...], dn,
                            preferred_element_type=jnp.float32)
    z = z + fc0_b_ref[...]
    noise = jnp.maximum(z, 0.0) + jnp.log1p(jnp.exp(-jnp.abs(z)))
    nmean = jnp.sum(noise, axis=1, keepdims=True) * (1.0 / E)
    dev = noise - nmean
    std = jnp.sqrt(jnp.sum(dev * dev, axis=1, keepdims=True) * (1.0 / (E - 1)))
    s = g + dev / std

    lane = jax.lax.broadcasted_iota(jnp.int32, (B, E), 1)
    rank = jnp.zeros((B, E), jnp.int32)
    for j in range(E):
        sj = s[:, j:j + 1]
        beats = (sj > s) | ((sj == s) & (j < lane))
        rank = rank + beats.astype(jnp.int32)
    mask = rank < TOP_K

    neg = jnp.float32(-1e30)
    gm = jnp.where(mask, g, neg)
    m = jnp.max(gm, axis=1, keepdims=True)
    ex = jnp.where(mask, jnp.exp(g - m), 0.0)
    cof_ref[...] = ex / jnp.sum(ex, axis=1, keepdims=True)

    cols = []
    for k in range(TOP_K):
        sel = (rank == k).astype(jnp.int32)
        cols.append(jnp.sum(sel * lane, axis=1, keepdims=True))
    idx_ref[...] = jnp.concatenate(cols, axis=1)


def _gate(pooled, fc0_w, fc0_b, fc1_w, fc1_b):
    return pl.pallas_call(
        _gate_body,
        in_specs=[
            pl.BlockSpec((B, C), lambda: (0, 0)),
            pl.BlockSpec((E, C), lambda: (0, 0)),
            pl.BlockSpec((1, E), lambda: (0, 0)),
            pl.BlockSpec((E, C), lambda: (0, 0)),
            pl.BlockSpec((1, E), lambda: (0, 0)),
        ],
        out_specs=[
            pl.BlockSpec((B, E), lambda: (0, 0)),
            pl.BlockSpec((B, TOP_K), lambda: (0, 0)),
        ],
        out_shape=[
            jax.ShapeDtypeStruct((B, E), jnp.float32),
            jax.ShapeDtypeStruct((B, TOP_K), jnp.int32),
        ],
    )(pooled, fc0_w, fc0_b.reshape(1, E), fc1_w, fc1_b.reshape(1, E))


def _stencil3x3(v, w, mL, mR, mT, mB):
    oL = jnp.where(mR, jnp.roll(v, 1, axis=1), 0.0)
    oR = jnp.where(mL, jnp.roll(v, -1, axis=1), 0.0)
    r0 = w[0] * oL + w[1] * v + w[2] * oR
    r1 = w[3] * oL + w[4] * v + w[5] * oR
    r2 = w[6] * oL + w[7] * v + w[8] * oR
    t0 = jnp.where(mT, jnp.roll(r0, 1, axis=0), 0.0)
    t2 = jnp.where(mB, jnp.roll(r2, -1, axis=0), 0.0)
    return t0 + r1 + t2


def _conv_body(idx_ref, x_ref, ew1_ref, eb1_ref, ew2_ref, eb2_ref, cof_ref,
               out_ref):
    i = pl.program_id(0)
    b = i // C

    col = jax.lax.broadcasted_iota(jnp.int32, (H, W), 1)
    row = jax.lax.broadcasted_iota(jnp.int32, (H, W), 0)
    mL = col != (W - 1)
    mR = col != 0
    mT = row != 0
    mB = row != (H - 1)

    xv = x_ref[0]
    acc = jnp.zeros((H, W), jnp.float32)
    for k in range(TOP_K):
        e = idx_ref[b, k]
        w1 = [ew1_ref[e, 0, 0, t] for t in range(9)]
        b1 = eb1_ref[e, 0, 0, 0]
        h = _stencil3x3(xv, w1, mL, mR, mT, mB)
        h = jnp.maximum(h + b1, 0.0)
        w2 = [ew2_ref[e, 0, 0, t] for t in range(9)]
        b2 = eb2_ref[e, 0, 0, 0]
        y = _stencil3x3(h, w2, mL, mR, mT, mB) + b2
        acc = acc + cof_ref[b, e] * y
    out_ref[0] = acc


def _conv(idx, x3, ew1v, eb1v, ew2v, eb2v, cof):
    def widx(i, idx_ref):
        return (0, i % C, 0, 0)

    grid_spec = pltpu.PrefetchScalarGridSpec(
        num_scalar_prefetch=1,
        grid=(NBC,),
        in_specs=[
            pl.BlockSpec((1, H, W), lambda i, idx_ref: (i, 0, 0)),
            pl.BlockSpec((E, 1, 1, 9), widx),
            pl.BlockSpec((E, 1, 1, 1), widx),
            pl.BlockSpec((E, 1, 1, 9), widx),
            pl.BlockSpec((E, 1, 1, 1), widx),
            pl.BlockSpec(memory_space=pltpu.SMEM),
        ],
        out_specs=pl.BlockSpec((1, H, W), lambda i, idx_ref: (i, 0, 0)),
    )
    return pl.pallas_call(
        _conv_body,
        grid_spec=grid_spec,
        out_shape=jax.ShapeDtypeStruct((NBC, H, W), jnp.float32),
    )(idx, x3, ew1v, eb1v, ew2v, eb2v, cof)


@jax.jit
def kernel(x, fc0_w, fc0_b, fc1_w, fc1_b, ew1, eb1, ew2, eb2):
    x2 = x.reshape(NBC, HW)
    pooled = _pool(x2).reshape(B, C)
    cof, idx = _gate(pooled, fc0_w, fc0_b, fc1_w, fc1_b)

    x3 = x.reshape(NBC, H, W)
    ew1v = ew1.reshape(E, C, 1, 9)
    eb1v = eb1.reshape(E, C, 1, 1)
    ew2v = ew2.reshape(E, C, 1, 9)
    eb2v = eb2.reshape(E, C, 1, 1)
    out = _conv(idx, x3, ew1v, eb1v, ew2v, eb2v, cof)
    return out.reshape(B, C, H, W)


# speedup vs baseline: 3.8431x; 1.0287x over previous
import functools

import jax
import jax.numpy as jnp
from jax.experimental import pallas as pl
from jax.experimental.pallas import tpu as pltpu

E = 6
TOP_K = 3
B, C, H, W = 2, 192, 224, 224
HW = H * W
NBC = B * C


def _pool_body(x_ref, out_ref):
    xb = x_ref[...]
    mx = jnp.max(xb, axis=1, keepdims=True)
    sm = jnp.sum(xb, axis=1, keepdims=True)
    out_ref[...] = mx + sm * (1.0 / HW)


def _pool(x2):
    rows = 48
    grid = (NBC // rows,)
    return pl.pallas_call(
        _pool_body,
        grid=grid,
        in_specs=[pl.BlockSpec((rows, HW), lambda i: (i, 0))],
        out_specs=pl.BlockSpec((rows, 1), lambda i: (i, 0)),
        out_shape=jax.ShapeDtypeStruct((NBC, 1), jnp.float32),
    )(x2)


def _gate_body(pooled_ref, fc0_w_ref, fc0_b_ref, fc1_w_ref, fc1_b_ref,
               cof_ref, idx_ref):
    pooled = pooled_ref[...]
    dn = (((1,), (1,)), ((), ()))
    g = jax.lax.dot_general(pooled, fc1_w_ref[...], dn,
                            preferred_element_type=jnp.float32)
    g = g + fc1_b_ref[...]
    g = jnp.where(g > 0, g, 0.2 * g)

    z = jax.lax.dot_general(pooled, fc0_w_ref[...], dn,
                            preferred_element_type=jnp.float32)
    z = z + fc0_b_ref[...]
    noise = jnp.maximum(z, 0.0) + jnp.log1p(jnp.exp(-jnp.abs(z)))
    nmean = jnp.sum(noise, axis=1, keepdims=True) * (1.0 / E)
    dev = noise - nmean
    std = jnp.sqrt(jnp.sum(dev * dev, axis=1, keepdims=True) * (1.0 / (E - 1)))
    s = g + dev / std

    lane = jax.lax.broadcasted_iota(jnp.int32, (B, E), 1)
    rank = jnp.zeros((B, E), jnp.int32)
    for j in range(E):
        sj = s[:, j:j + 1]
        beats = (sj > s) | ((sj == s) & (j < lane))
        rank = rank + beats.astype(jnp.int32)
    mask = rank < TOP_K

    neg = jnp.float32(-1e30)
    gm = jnp.where(mask, g, neg)
    m = jnp.max(gm, axis=1, keepdims=True)
    ex = jnp.where(mask, jnp.exp(g - m), 0.0)
    cof_ref[...] = ex / jnp.sum(ex, axis=1, keepdims=True)

    cols = []
    for k in range(TOP_K):
        sel = (rank == k).astype(jnp.int32)
        cols.append(jnp.sum(sel * lane, axis=1, keepdims=True))
    idx_ref[...] = jnp.concatenate(cols, axis=1)


def _gate(pooled, fc0_w, fc0_b, fc1_w, fc1_b):
    return pl.pallas_call(
        _gate_body,
        in_specs=[
            pl.BlockSpec((B, C), lambda: (0, 0)),
            pl.BlockSpec((E, C), lambda: (0, 0)),
            pl.BlockSpec((1, E), lambda: (0, 0)),
            pl.BlockSpec((E, C), lambda: (0, 0)),
            pl.BlockSpec((1, E), lambda: (0, 0)),
        ],
        out_specs=[
            pl.BlockSpec((B, E), lambda: (0, 0)),
            pl.BlockSpec((B, TOP_K), lambda: (0, 0)),
        ],
        out_shape=[
            jax.ShapeDtypeStruct((B, E), jnp.float32),
            jax.ShapeDtypeStruct((B, TOP_K), jnp.int32),
        ],
    )(pooled, fc0_w, fc0_b.reshape(1, E), fc1_w, fc1_b.reshape(1, E))


CB = 4


def _hrows(oL, v, oR, w):
    r0 = w[0] * oL + w[1] * v + w[2] * oR
    r1 = w[3] * oL + w[4] * v + w[5] * oR
    r2 = w[6] * oL + w[7] * v + w[8] * oR
    return r0, r1, r2


def _vcomb(r0, r1, r2, mT, mB):
    t0 = jnp.where(mT, jnp.roll(r0, 1, axis=0), 0.0)
    t2 = jnp.where(mB, jnp.roll(r2, -1, axis=0), 0.0)
    return t0 + r1 + t2


def _conv_body(idx_ref, x_ref, ew1_ref, eb1_ref, ew2_ref, eb2_ref, cof_ref,
               out_ref):
    i = pl.program_id(0)
    b = i // (C // CB)

    col = jax.lax.broadcasted_iota(jnp.int32, (H, W), 1)
    row = jax.lax.broadcasted_iota(jnp.int32, (H, W), 0)
    mL = col != (W - 1)
    mR = col != 0
    mT = row != 0
    mB = row != (H - 1)

    for ci in range(CB):
        xv = x_ref[ci]
        xL = jnp.where(mR, jnp.roll(xv, 1, axis=1), 0.0)
        xR = jnp.where(mL, jnp.roll(xv, -1, axis=1), 0.0)
        acc = jnp.zeros((H, W), jnp.float32)
        for k in range(TOP_K):
            e = idx_ref[b, k]
            coef = cof_ref[b, e]
            w1 = [ew1_ref[e, ci, 0, t] for t in range(9)]
            b1 = eb1_ref[e, ci, 0, 0]
            h = _vcomb(*_hrows(xL, xv, xR, w1), mT, mB)
            h = jnp.maximum(h + b1, 0.0)
            w2 = [coef * ew2_ref[e, ci, 0, t] for t in range(9)]
            b2 = coef * eb2_ref[e, ci, 0, 0]
            hL = jnp.where(mR, jnp.roll(h, 1, axis=1), 0.0)
            hR = jnp.where(mL, jnp.roll(h, -1, axis=1), 0.0)
            y = _vcomb(*_hrows(hL, h, hR, w2), mT, mB)
            acc = acc + (y + b2)
        out_ref[ci] = acc


def _conv(idx, x3, ew1v, eb1v, ew2v, eb2v, cof):
    def widx(i, idx_ref):
        return (0, i % (C // CB), 0, 0)

    grid_spec = pltpu.PrefetchScalarGridSpec(
        num_scalar_prefetch=1,
        grid=(NBC // CB,),
        in_specs=[
            pl.BlockSpec((CB, H, W), lambda i, idx_ref: (i, 0, 0)),
            pl.BlockSpec((E, CB, 1, 9), widx),
            pl.BlockSpec((E, CB, 1, 1), widx),
            pl.BlockSpec((E, CB, 1, 9), widx),
            pl.BlockSpec((E, CB, 1, 1), widx),
            pl.BlockSpec(memory_space=pltpu.SMEM),
        ],
        out_specs=pl.BlockSpec((CB, H, W), lambda i, idx_ref: (i, 0, 0)),
    )
    return pl.pallas_call(
        _conv_body,
        grid_spec=grid_spec,
        out_shape=jax.ShapeDtypeStruct((NBC, H, W), jnp.float32),
    )(idx, x3, ew1v, eb1v, ew2v, eb2v, cof)


@jax.jit
def kernel(x, fc0_w, fc0_b, fc1_w, fc1_b, ew1, eb1, ew2, eb2):
    x2 = x.reshape(NBC, HW)
    pooled = _pool(x2).reshape(B, C)
    cof, idx = _gate(pooled, fc0_w, fc0_b, fc1_w, fc1_b)

    x3 = x.reshape(NBC, H, W)
    ew1v = ew1.reshape(E, C, 1, 9)
    eb1v = eb1.reshape(E, C, 1, 1)
    ew2v = ew2.reshape(E, C, 1, 9)
    eb2v = eb2.reshape(E, C, 1, 1)
    out = _conv(idx, x3, ew1v, eb1v, ew2v, eb2v, cof)
    return out.reshape(B, C, H, W)


# speedup vs baseline: 3.8667x; 1.0061x over previous
import functools

import jax
import jax.numpy as jnp
from jax.experimental import pallas as pl
from jax.experimental.pallas import tpu as pltpu

E = 6
TOP_K = 3
B, C, H, W = 2, 192, 224, 224
HW = H * W
NBC = B * C


def _pool_body(x_ref, out_ref):
    xb = x_ref[...]
    mx = jnp.max(xb, axis=1, keepdims=True)
    sm = jnp.sum(xb, axis=1, keepdims=True)
    out_ref[...] = mx + sm * (1.0 / HW)


def _pool(x2):
    rows = 48
    grid = (NBC // rows,)
    return pl.pallas_call(
        _pool_body,
        grid=grid,
        in_specs=[pl.BlockSpec((rows, HW), lambda i: (i, 0))],
        out_specs=pl.BlockSpec((rows, 1), lambda i: (i, 0)),
        out_shape=jax.ShapeDtypeStruct((NBC, 1), jnp.float32),
    )(x2)


def _gate_body(pooled_ref, fc0_w_ref, fc0_b_ref, fc1_w_ref, fc1_b_ref,
               cof_ref, idx_ref):
    pooled = pooled_ref[...]
    dn = (((1,), (1,)), ((), ()))
    g = jax.lax.dot_general(pooled, fc1_w_ref[...], dn,
                            preferred_element_type=jnp.float32)
    g = g + fc1_b_ref[...]
    g = jnp.where(g > 0, g, 0.2 * g)

    z = jax.lax.dot_general(pooled, fc0_w_ref[...], dn,
                            preferred_element_type=jnp.float32)
    z = z + fc0_b_ref[...]
    noise = jnp.maximum(z, 0.0) + jnp.log1p(jnp.exp(-jnp.abs(z)))
    nmean = jnp.sum(noise, axis=1, keepdims=True) * (1.0 / E)
    dev = noise - nmean
    std = jnp.sqrt(jnp.sum(dev * dev, axis=1, keepdims=True) * (1.0 / (E - 1)))
    s = g + dev / std

    lane = jax.lax.broadcasted_iota(jnp.int32, (B, E), 1)
    rank = jnp.zeros((B, E), jnp.int32)
    for j in range(E):
        sj = s[:, j:j + 1]
        beats = (sj > s) | ((sj == s) & (j < lane))
        rank = rank + beats.astype(jnp.int32)
    mask = rank < TOP_K

    neg = jnp.float32(-1e30)
    gm = jnp.where(mask, g, neg)
    m = jnp.max(gm, axis=1, keepdims=True)
    ex = jnp.where(mask, jnp.exp(g - m), 0.0)
    cof_ref[...] = ex / jnp.sum(ex, axis=1, keepdims=True)

    cols = []
    for k in range(TOP_K):
        sel = (rank == k).astype(jnp.int32)
        cols.append(jnp.sum(sel * lane, axis=1, keepdims=True))
    idx_ref[...] = jnp.concatenate(cols, axis=1)


def _gate(pooled, fc0_w, fc0_b, fc1_w, fc1_b):
    return pl.pallas_call(
        _gate_body,
        in_specs=[
            pl.BlockSpec((B, C), lambda: (0, 0)),
            pl.BlockSpec((E, C), lambda: (0, 0)),
            pl.BlockSpec((1, E), lambda: (0, 0)),
            pl.BlockSpec((E, C), lambda: (0, 0)),
            pl.BlockSpec((1, E), lambda: (0, 0)),
        ],
        out_specs=[
            pl.BlockSpec((B, E), lambda: (0, 0)),
            pl.BlockSpec((B, TOP_K), lambda: (0, 0)),
        ],
        out_shape=[
            jax.ShapeDtypeStruct((B, E), jnp.float32),
            jax.ShapeDtypeStruct((B, TOP_K), jnp.int32),
        ],
    )(pooled, fc0_w, fc0_b.reshape(1, E), fc1_w, fc1_b.reshape(1, E))


CB = 8


def _hrows(oL, v, oR, w):
    r0 = w[0] * oL + w[1] * v + w[2] * oR
    r1 = w[3] * oL + w[4] * v + w[5] * oR
    r2 = w[6] * oL + w[7] * v + w[8] * oR
    return r0, r1, r2


def _vcomb(r0, r1, r2, mT, mB):
    t0 = jnp.where(mT, jnp.roll(r0, 1, axis=0), 0.0)
    t2 = jnp.where(mB, jnp.roll(r2, -1, axis=0), 0.0)
    return t0 + r1 + t2


def _conv_body(idx_ref, x_ref, ew1_ref, eb1_ref, ew2_ref, eb2_ref, cof_ref,
               out_ref):
    i = pl.program_id(0)
    b = i // (C // CB)

    col = jax.lax.broadcasted_iota(jnp.int32, (H, W), 1)
    row = jax.lax.broadcasted_iota(jnp.int32, (H, W), 0)
    mL = col != (W - 1)
    mR = col != 0
    mT = row != 0
    mB = row != (H - 1)

    for ci in range(CB):
        xv = x_ref[ci]
        xL = jnp.where(mR, jnp.roll(xv, 1, axis=1), 0.0)
        xR = jnp.where(mL, jnp.roll(xv, -1, axis=1), 0.0)
        acc = jnp.zeros((H, W), jnp.float32)
        for k in range(TOP_K):
            e = idx_ref[b, k]
            coef = cof_ref[b, e]
            w1 = [ew1_ref[e, ci, 0, t] for t in range(9)]
            b1 = eb1_ref[e, ci, 0, 0]
            h = _vcomb(*_hrows(xL, xv, xR, w1), mT, mB)
            h = jnp.maximum(h + b1, 0.0)
            w2 = [coef * ew2_ref[e, ci, 0, t] for t in range(9)]
            b2 = coef * eb2_ref[e, ci, 0, 0]
            hL = jnp.where(mR, jnp.roll(h, 1, axis=1), 0.0)
            hR = jnp.where(mL, jnp.roll(h, -1, axis=1), 0.0)
            y = _vcomb(*_hrows(hL, h, hR, w2), mT, mB)
            acc = acc + (y + b2)
        out_ref[ci] = acc


def _conv(idx, x3, ew1v, eb1v, ew2v, eb2v, cof):
    def widx(i, idx_ref):
        return (0, i % (C // CB), 0, 0)

    grid_spec = pltpu.PrefetchScalarGridSpec(
        num_scalar_prefetch=1,
        grid=(NBC // CB,),
        in_specs=[
            pl.BlockSpec((CB, H, W), lambda i, idx_ref: (i, 0, 0)),
            pl.BlockSpec((E, CB, 1, 9), widx),
            pl.BlockSpec((E, CB, 1, 1), widx),
            pl.BlockSpec((E, CB, 1, 9), widx),
            pl.BlockSpec((E, CB, 1, 1), widx),
            pl.BlockSpec(memory_space=pltpu.SMEM),
        ],
        out_specs=pl.BlockSpec((CB, H, W), lambda i, idx_ref: (i, 0, 0)),
    )
    return pl.pallas_call(
        _conv_body,
        grid_spec=grid_spec,
        out_shape=jax.ShapeDtypeStruct((NBC, H, W), jnp.float32),
    )(idx, x3, ew1v, eb1v, ew2v, eb2v, cof)


@jax.jit
def kernel(x, fc0_w, fc0_b, fc1_w, fc1_b, ew1, eb1, ew2, eb2):
    x2 = x.reshape(NBC, HW)
    pooled = _pool(x2).reshape(B, C)
    cof, idx = _gate(pooled, fc0_w, fc0_b, fc1_w, fc1_b)

    x3 = x.reshape(NBC, H, W)
    ew1v = ew1.reshape(E, C, 1, 9)
    eb1v = eb1.reshape(E, C, 1, 1)
    ew2v = ew2.reshape(E, C, 1, 9)
    eb2v = eb2.reshape(E, C, 1, 1)
    out = _conv(idx, x3, ew1v, eb1v, ew2v, eb2v, cof)
    return out.reshape(B, C, H, W)


# speedup vs baseline: 4.1919x; 1.0841x over previous
import functools

import jax
import jax.numpy as jnp
from jax.experimental import pallas as pl
from jax.experimental.pallas import tpu as pltpu

E = 6
TOP_K = 3
B, C, H, W = 2, 192, 224, 224
HW = H * W
NBC = B * C


def _pool_body(x_ref, out_ref):
    xb = x_ref[...]
    mx = jnp.max(xb, axis=1, keepdims=True)
    sm = jnp.sum(xb, axis=1, keepdims=True)
    out_ref[...] = mx + sm * (1.0 / HW)


def _pool(x2):
    rows = 48
    grid = (NBC // rows,)
    return pl.pallas_call(
        _pool_body,
        grid=grid,
        in_specs=[pl.BlockSpec((rows, HW), lambda i: (i, 0))],
        out_specs=pl.BlockSpec((rows, 1), lambda i: (i, 0)),
        out_shape=jax.ShapeDtypeStruct((NBC, 1), jnp.float32),
    )(x2)


def _gate_body(pooled_ref, fc0_w_ref, fc0_b_ref, fc1_w_ref, fc1_b_ref,
               cof_ref, idx_ref):
    pooled = pooled_ref[...]
    dn = (((1,), (1,)), ((), ()))
    g = jax.lax.dot_general(pooled, fc1_w_ref[...], dn,
                            preferred_element_type=jnp.float32)
    g = g + fc1_b_ref[...]
    g = jnp.where(g > 0, g, 0.2 * g)

    z = jax.lax.dot_general(pooled, fc0_w_ref[...], dn,
                            preferred_element_type=jnp.float32)
    z = z + fc0_b_ref[...]
    noise = jnp.maximum(z, 0.0) + jnp.log1p(jnp.exp(-jnp.abs(z)))
    nmean = jnp.sum(noise, axis=1, keepdims=True) * (1.0 / E)
    dev = noise - nmean
    std = jnp.sqrt(jnp.sum(dev * dev, axis=1, keepdims=True) * (1.0 / (E - 1)))
    s = g + dev / std

    lane = jax.lax.broadcasted_iota(jnp.int32, (B, E), 1)
    rank = jnp.zeros((B, E), jnp.int32)
    for j in range(E):
        sj = s[:, j:j + 1]
        beats = (sj > s) | ((sj == s) & (j < lane))
        rank = rank + beats.astype(jnp.int32)
    mask = rank < TOP_K

    neg = jnp.float32(-1e30)
    gm = jnp.where(mask, g, neg)
    m = jnp.max(gm, axis=1, keepdims=True)
    ex = jnp.where(mask, jnp.exp(g - m), 0.0)
    cof_ref[...] = ex / jnp.sum(ex, axis=1, keepdims=True)

    cols = []
    for k in range(TOP_K):
        sel = (rank == k).astype(jnp.int32)
        cols.append(jnp.sum(sel * lane, axis=1, keepdims=True))
    idx_ref[...] = jnp.concatenate(cols, axis=1)


def _gate(pooled, fc0_w, fc0_b, fc1_w, fc1_b):
    return pl.pallas_call(
        _gate_body,
        in_specs=[
            pl.BlockSpec((B, C), lambda: (0, 0)),
            pl.BlockSpec((E, C), lambda: (0, 0)),
            pl.BlockSpec((1, E), lambda: (0, 0)),
            pl.BlockSpec((E, C), lambda: (0, 0)),
            pl.BlockSpec((1, E), lambda: (0, 0)),
        ],
        out_specs=[
            pl.BlockSpec((B, E), lambda: (0, 0)),
            pl.BlockSpec((B, TOP_K), lambda: (0, 0)),
        ],
        out_shape=[
            jax.ShapeDtypeStruct((B, E), jnp.float32),
            jax.ShapeDtypeStruct((B, TOP_K), jnp.int32),
        ],
    )(pooled, fc0_w, fc0_b.reshape(1, E), fc1_w, fc1_b.reshape(1, E))


CB = 8


HP = H + 8
WP = W + 32


def _stencil(v, w):
    oL = jnp.roll(v, 1, axis=1)
    oR = jnp.roll(v, -1, axis=1)
    r0 = w[0] * oL + w[1] * v + w[2] * oR
    r1 = w[3] * oL + w[4] * v + w[5] * oR
    r2 = w[6] * oL + w[7] * v + w[8] * oR
    return jnp.roll(r0, 1, axis=0) + r1 + jnp.roll(r2, -1, axis=0)


def _conv_body(idx_ref, x_ref, ew1_ref, eb1_ref, ew2_ref, eb2_ref, cof_ref,
               out_ref, xg):
    i = pl.program_id(0)
    b = i // (C // CB)

    @pl.when(i == 0)
    def _init():
        xg[...] = jnp.zeros_like(xg)

    col = jax.lax.broadcasted_iota(jnp.int32, (HP, WP), 1)
    row = jax.lax.broadcasted_iota(jnp.int32, (HP, WP), 0)
    interior = (row >= 1) & (row <= H) & (col >= 1) & (col <= W)

    for ci in range(CB):
        xg[1:H + 1, 1:W + 1] = x_ref[ci]
        xv = xg[...]
        acc = None
        bsum = jnp.float32(0.0)
        for k in range(TOP_K):
            e = idx_ref[b, k]
            coef = cof_ref[b, e]
            w1 = [ew1_ref[e, ci, 0, t] for t in range(9)]
            b1 = eb1_ref[e, ci, 0, 0]
            h = jnp.maximum(_stencil(xv, w1) + b1, 0.0)
            h = jnp.where(interior, h, 0.0)
            w2 = [coef * ew2_ref[e, ci, 0, t] for t in range(9)]
            bsum = bsum + coef * eb2_ref[e, ci, 0, 0]
            y = _stencil(h, w2)
            acc = y if acc is None else acc + y
        out_ref[ci] = acc[1:H + 1, 1:W + 1] + bsum


def _conv(idx, x3, ew1v, eb1v, ew2v, eb2v, cof):
    def widx(i, idx_ref):
        return (0, i % (C // CB), 0, 0)

    grid_spec = pltpu.PrefetchScalarGridSpec(
        num_scalar_prefetch=1,
        grid=(NBC // CB,),
        in_specs=[
            pl.BlockSpec((CB, H, W), lambda i, idx_ref: (i, 0, 0)),
            pl.BlockSpec((E, CB, 1, 9), widx),
            pl.BlockSpec((E, CB, 1, 1), widx),
            pl.BlockSpec((E, CB, 1, 9), widx),
            pl.BlockSpec((E, CB, 1, 1), widx),
            pl.BlockSpec(memory_space=pltpu.SMEM),
        ],
        out_specs=pl.BlockSpec((CB, H, W), lambda i, idx_ref: (i, 0, 0)),
        scratch_shapes=[pltpu.VMEM((HP, WP), jnp.float32)],
    )
    return pl.pallas_call(
        _conv_body,
        grid_spec=grid_spec,
        out_shape=jax.ShapeDtypeStruct((NBC, H, W), jnp.float32),
    )(idx, x3, ew1v, eb1v, ew2v, eb2v, cof)


@jax.jit
def kernel(x, fc0_w, fc0_b, fc1_w, fc1_b, ew1, eb1, ew2, eb2):
    x2 = x.reshape(NBC, HW)
    pooled = _pool(x2).reshape(B, C)
    cof, idx = _gate(pooled, fc0_w, fc0_b, fc1_w, fc1_b)

    x3 = x.reshape(NBC, H, W)
    ew1v = ew1.reshape(E, C, 1, 9)
    eb1v = eb1.reshape(E, C, 1, 1)
    ew2v = ew2.reshape(E, C, 1, 9)
    eb2v = eb2.reshape(E, C, 1, 1)
    out = _conv(idx, x3, ew1v, eb1v, ew2v, eb2v, cof)
    return out.reshape(B, C, H, W)
